# baseline (speedup 1.0000x reference)
import jax
import jax.numpy as jnp
from jax.experimental import pallas as pl
from jax.experimental.pallas import tpu as pltpu

E = 8
TILE = 4096


def _route(lt):
    e_iota = jax.lax.broadcasted_iota(jnp.int32, lt.shape, 0)
    m1 = jnp.max(lt, axis=0, keepdims=True)
    i1 = jnp.min(jnp.where(lt == m1, e_iota, E), axis=0, keepdims=True)
    masked = jnp.where(e_iota == i1, -jnp.inf, lt)
    m2 = jnp.max(masked, axis=0, keepdims=True)
    i2 = jnp.min(jnp.where(masked == m2, e_iota, E), axis=0, keepdims=True)

    z = jnp.sum(jnp.exp(lt - m1), axis=0, keepdims=True)
    p1 = 1.0 / z
    p2 = jnp.exp(m2 - m1) * p1
    w1 = jax.nn.sigmoid(p1 - p2)
    wout = jnp.concatenate([w1, 1.0 - w1], axis=0)
    iout = jnp.concatenate([i1, i2], axis=0)
    return wout, iout


def _router_body(x_ref, w_ref, wout_ref, iout_ref, lt_ref):
    i = pl.program_id(0)
    n = pl.num_programs(0)

    @pl.when(i < n - 1)
    def _matmul():
        logits = jnp.dot(
            x_ref[...], w_ref[...], preferred_element_type=jnp.float32
        )
        lt_ref[i % 2] = logits.T

    @pl.when(i > 0)
    def _routing():
        wout, iout = _route(lt_ref[(i - 1) % 2])
        wout_ref[...] = wout
        iout_ref[...] = iout


def kernel(x, kernel_DE):
    B, T, D = x.shape
    N = B * T
    nb = N // TILE
    xf = x.reshape(N, D)
    wout, iout = pl.pallas_call(
        _router_body,
        grid=(nb + 1,),
        in_specs=[
            pl.BlockSpec((TILE, D), lambda i: (jnp.minimum(i, nb - 1), 0)),
            pl.BlockSpec((D, E), lambda i: (0, 0)),
        ],
        out_specs=[
            pl.BlockSpec((2, TILE), lambda i: (0, jnp.maximum(i - 1, 0))),
            pl.BlockSpec((2, TILE), lambda i: (0, jnp.maximum(i - 1, 0))),
        ],
        out_shape=[
            jax.ShapeDtypeStruct((2, N), jnp.float32),
            jax.ShapeDtypeStruct((2, N), jnp.int32),
        ],
        scratch_shapes=[pltpu.VMEM((2, E, TILE), jnp.float32)],
    )(xf, kernel_DE)
    return wout.T.reshape(B, T, 2), iout.T.reshape(B, T, 2)


# speedup vs baseline: 1.0181x; 1.0181x over previous
import jax
import jax.numpy as jnp
from jax.experimental import pallas as pl

E = 8
TILE = 4096


def _router_body(x_ref, w_ref, wout_ref, iout_ref):
    logits = jnp.dot(x_ref[...], w_ref[...], preferred_element_type=jnp.float32)
    lt = logits.T

    e_iota = jax.lax.broadcasted_iota(jnp.int32, lt.shape, 0)
    m1 = jnp.max(lt, axis=0, keepdims=True)
    i1 = jnp.min(jnp.where(lt == m1, e_iota, E), axis=0, keepdims=True)
    masked = jnp.where(e_iota == i1, -jnp.inf, lt)
    m2 = jnp.max(masked, axis=0, keepdims=True)
    i2 = jnp.min(jnp.where(masked == m2, e_iota, E), axis=0, keepdims=True)

    z = jnp.sum(jnp.exp(lt - m1), axis=0, keepdims=True)
    p1 = 1.0 / z
    p2 = jnp.exp(m2 - m1) * p1
    w1 = jax.nn.sigmoid(p1 - p2)

    wout_ref[...] = jnp.concatenate([w1, 1.0 - w1], axis=0)
    iout_ref[...] = jnp.concatenate([i1, i2], axis=0)


def kernel(x, kernel_DE):
    B, T, D = x.shape
    N = B * T
    xf = x.reshape(N, D)
    wout, iout = pl.pallas_call(
        _router_body,
        grid=(N // TILE,),
        in_specs=[
            pl.BlockSpec((TILE, D), lambda i: (i, 0)),
            pl.BlockSpec((D, E), lambda i: (0, 0)),
        ],
        out_specs=[
            pl.BlockSpec((2, TILE), lambda i: (0, i)),
            pl.BlockSpec((2, TILE), lambda i: (0, i)),
        ],
        out_shape=[
            jax.ShapeDtypeStruct((2, N), jnp.float32),
            jax.ShapeDtypeStruct((2, N), jnp.int32),
        ],
    )(xf, kernel_DE)
    return wout.T.reshape(B, T, 2), iout.T.reshape(B, T, 2)
